# baseline (speedup 1.0000x reference)
import functools
import jax
import jax.numpy as jnp
from jax.experimental import pallas as pl
from jax.experimental.pallas import tpu as pltpu

_NCH = 4


def _pass1_body(x_ref, wq_ref, bq_ref, wk_ref, bk_ref, we_ref, attr_ref,
                idx_ref, alpha_ref, den_ref, qs, ks, tq, *, GP, C, ED):
    h = pl.program_id(0)
    c = pl.program_id(1)

    @pl.when(c == 0)
    def _():
        qs[...] = jnp.dot(x_ref[...], wq_ref[...]) + bq_ref[pl.ds(h, 1), :]
        ks[...] = jnp.dot(x_ref[...], wk_ref[...]) + bk_ref[pl.ds(h, 1), :]
        tq[...] = jax.lax.dot_general(qs[...], we_ref[...],
                                      (((1,), (1,)), ((), ())))
        den_ref[0] = jnp.zeros_like(den_ref[0])

    scale = 1.0 / (C ** 0.5)
    lane = jax.lax.broadcasted_iota(jnp.int32, (1, 128), 1) // 16

    def body(g, carry):
        row = idx_ref[pl.ds(g, 1), :]
        arow = attr_ref[pl.ds(g, 1), :]
        out_row = jnp.zeros((1, 128), jnp.float32)
        for j in range(8):
            s = row[0, 2 * j]
            d = row[0, 2 * j + 1]
            qrow = qs[pl.ds(d, 1), :]
            krow = ks[pl.ds(s, 1), :]
            tqr = tq[pl.ds(d, 1), :]
            aj = arow[:, 16 * j:16 * j + ED]
            logit = (jnp.sum(qrow * krow) + jnp.sum(tqr * aj)) * scale
            ex = jnp.exp(logit)
            out_row = out_row + jnp.where(lane == j, ex, 0.0)
            den_ref[0, pl.ds(d, 1), :] = den_ref[0, pl.ds(d, 1), :] + ex
        alpha_ref[0, pl.ds(g, 1), :] = out_row
        return carry

    jax.lax.fori_loop(0, GP, body, 0)


def _pass2_body(x_ref, wv_ref, bv_ref, we_ref, attr_ref, idx_ref,
                alpha_ref, den_ref, out_ref, vs, ss, *, GP, H, ED):
    h = pl.program_id(0)
    c = pl.program_id(1)

    @pl.when(c == 0)
    def _():
        vs[...] = jnp.dot(x_ref[...], wv_ref[...]) + bv_ref[pl.ds(h, 1), :]
        ss[...] = jnp.zeros_like(ss)

    @pl.when(jnp.logical_and(h == 0, c == 0))
    def _():
        out_ref[...] = jnp.zeros_like(out_ref)

    inv_h = 1.0 / H

    def body(g, carry):
        row = idx_ref[pl.ds(g, 1), :]
        arow = attr_ref[pl.ds(g, 1), :]
        alpha_row = alpha_ref[0, pl.ds(g, 1), :]
        for j in range(8):
            s = row[0, 2 * j]
            d = row[0, 2 * j + 1]
            ex = alpha_row[0, 16 * j]
            dn = den_ref[0, pl.ds(d, 1), 0:1]
            a = (ex / (dn[0, 0] + 1e-16)) * inv_h
            vrow = vs[pl.ds(s, 1), :]
            aj = arow[:, 16 * j:16 * j + ED]
            out_ref[pl.ds(d, 1), :] = out_ref[pl.ds(d, 1), :] + a * vrow
            ss[pl.ds(d, 1), :] = ss[pl.ds(d, 1), :] + a * aj
        return carry

    jax.lax.fori_loop(0, GP, body, 0)

    @pl.when(c == _NCH - 1)
    def _():
        out_ref[...] = out_ref[...] + jnp.dot(ss[...], we_ref[...])


def _norm_body(aggr_ref, x_ref, ws_ref, bs_ref, batch_ref, gw_ref, gb_ref,
               gms_ref, out_ref, *, G):
    z = aggr_ref[...] + jnp.dot(x_ref[...], ws_ref[...]) + bs_ref[0:1, :]
    n = z.shape[0]
    b = batch_ref[:, 0:1]
    iota = jax.lax.broadcasted_iota(jnp.int32, (n, G), 1)
    oh = (iota == b).astype(jnp.float32)
    cnt = jnp.maximum(jnp.sum(oh, axis=0, keepdims=True), 1.0)
    tot = jax.lax.dot_general(oh, z, (((0,), (0,)), ((), ())))
    mean = tot / cnt.T
    xm = z - jnp.dot(oh, mean) * gms_ref[0:1, :]
    var = jax.lax.dot_general(oh, xm * xm, (((0,), (0,)), ((), ()))) / cnt.T
    std = jnp.sqrt(var + 1e-5)
    y = gw_ref[0:1, :] * xm / jnp.dot(oh, std) + gb_ref[0:1, :]
    out_ref[...] = jnp.where(y >= 0, y, 0.01 * y)


def _layer(x, attr_pk, idx_pk, batch_pad, Wq, bq, Wk, bk, Wv, bv, We, Ws, bs,
           gw, gb, gms, ED):
    N, D = x.shape
    NG = attr_pk.shape[0]
    GP = NG // _NCH
    HC = Wq.shape[1]
    C = Ws.shape[1]
    H = HC // C
    G = 64

    bq2 = bq.reshape(H, C)
    bk2 = bk.reshape(H, C)
    bv2 = bv.reshape(H, C)

    head_w = pl.BlockSpec((D, C), lambda h, c: (0, h))
    head_we = pl.BlockSpec((ED, C), lambda h, c: (0, h))
    full = lambda a: pl.BlockSpec(a.shape, lambda h, c: (0,) * a.ndim)
    chunk = pl.BlockSpec((GP, 128), lambda h, c: (c, 0))

    alpha, den = pl.pallas_call(
        functools.partial(_pass1_body, GP=GP, C=C, ED=ED),
        grid=(H, _NCH),
        in_specs=[full(x), head_w, full(bq2), head_w, full(bk2), head_we,
                  chunk, chunk],
        out_specs=[pl.BlockSpec((1, GP, 128), lambda h, c: (h, c, 0)),
                   pl.BlockSpec((1, N, 8), lambda h, c: (h, 0, 0))],
        out_shape=[jax.ShapeDtypeStruct((H, NG, 128), jnp.float32),
                   jax.ShapeDtypeStruct((H, N, 8), jnp.float32)],
        scratch_shapes=[pltpu.VMEM((N, C), jnp.float32),
                        pltpu.VMEM((N, C), jnp.float32),
                        pltpu.VMEM((N, ED), jnp.float32)],
    )(x, Wq, bq2, Wk, bk2, We, attr_pk, idx_pk)

    aggr = pl.pallas_call(
        functools.partial(_pass2_body, GP=GP, H=H, ED=ED),
        grid=(H, _NCH),
        in_specs=[full(x), head_w, full(bv2), head_we, chunk, chunk,
                  pl.BlockSpec((1, GP, 128), lambda h, c: (h, c, 0)),
                  pl.BlockSpec((1, N, 8), lambda h, c: (h, 0, 0))],
        out_specs=pl.BlockSpec((N, C), lambda h, c: (0, 0)),
        out_shape=jax.ShapeDtypeStruct((N, C), jnp.float32),
        scratch_shapes=[pltpu.VMEM((N, C), jnp.float32),
                        pltpu.VMEM((N, ED), jnp.float32)],
    )(x, Wv, bv2, We, attr_pk, idx_pk, alpha, den)

    vec = lambda v: jnp.broadcast_to(v.reshape(1, -1), (8, v.shape[0]))
    out = pl.pallas_call(
        functools.partial(_norm_body, G=G),
        out_shape=jax.ShapeDtypeStruct((N, C), jnp.float32),
    )(aggr, x, Ws, vec(bs), batch_pad, vec(gw), vec(gb), vec(gms))
    return out


def kernel(x, index, attr, batch, params):
    E = index.shape[1]
    N = x.shape[0]
    ED = attr.shape[1]
    attr_pk = attr.reshape(E // 8, 8 * ED)
    idx_pk = jnp.stack([index[0], index[1]], axis=1).reshape(E // 8, 16)
    batch_pad = jnp.broadcast_to(batch.reshape(N, 1), (N, 8)).astype(jnp.int32)

    h = x
    for l in ('1', '2'):
        h = _layer(h, attr_pk, idx_pk, batch_pad,
                   params['Wq' + l], params['bq' + l],
                   params['Wk' + l], params['bk' + l],
                   params['Wv' + l], params['bv' + l],
                   params['We' + l], params['Ws' + l], params['bs' + l],
                   params['gw' + l], params['gb' + l], params['gms' + l], ED)
    return h


# speedup vs baseline: 1.2942x; 1.2942x over previous
import functools
import jax
import jax.numpy as jnp
from jax.experimental import pallas as pl
from jax.experimental.pallas import tpu as pltpu

_NCH = 10


def _pass1_body(x_ref, wq_ref, bq_ref, wk_ref, bk_ref, we_ref, attr_ref,
                idx_ref, alpha_ref, den_ref, qs, ks, tq, *, GP, C, ED):
    hp = pl.program_id(0)
    c = pl.program_id(1)

    @pl.when(c == 0)
    def _():
        b2 = jnp.concatenate([bq_ref[pl.ds(2 * hp, 1), :],
                              bq_ref[pl.ds(2 * hp + 1, 1), :]], axis=1)
        qs[...] = jnp.dot(x_ref[...], wq_ref[...]) + b2
        b2k = jnp.concatenate([bk_ref[pl.ds(2 * hp, 1), :],
                               bk_ref[pl.ds(2 * hp + 1, 1), :]], axis=1)
        ks[...] = jnp.dot(x_ref[...], wk_ref[...]) + b2k
        tq[:, 0:ED] = jax.lax.dot_general(
            qs[:, 0:C], we_ref[:, 0:C], (((1,), (1,)), ((), ())))
        tq[:, ED:2 * ED] = jax.lax.dot_general(
            qs[:, C:2 * C], we_ref[:, C:2 * C], (((1,), (1,)), ((), ())))
        den_ref[0] = jnp.zeros_like(den_ref[0])

    scale = 1.0 / (C ** 0.5)
    lane = jax.lax.broadcasted_iota(jnp.int32, (1, 128), 1) // 16

    def body(g, carry):
        row = idx_ref[pl.ds(g, 1), :]
        arow = attr_ref[pl.ds(g, 1), :]
        row0 = jnp.zeros((1, 128), jnp.float32)
        row1 = jnp.zeros((1, 128), jnp.float32)
        for j in range(8):
            s = row[0, 2 * j]
            d = row[0, 2 * j + 1]
            qrow = qs[pl.ds(d, 1), :]
            krow = ks[pl.ds(s, 1), :]
            tqr = tq[pl.ds(d, 1), :]
            aj = arow[:, 16 * j:16 * j + ED]
            l0 = (jnp.sum(qrow[:, 0:C] * krow[:, 0:C])
                  + jnp.sum(tqr[:, 0:ED] * aj)) * scale
            l1 = (jnp.sum(qrow[:, C:2 * C] * krow[:, C:2 * C])
                  + jnp.sum(tqr[:, ED:2 * ED] * aj)) * scale
            e0 = jnp.exp(l0)
            e1 = jnp.exp(l1)
            row0 = row0 + jnp.where(lane == j, e0, 0.0)
            row1 = row1 + jnp.where(lane == j, e1, 0.0)
            den_ref[0, pl.ds(d, 1), 0:8] = den_ref[0, pl.ds(d, 1), 0:8] + e0
            den_ref[0, pl.ds(d, 1), 8:16] = den_ref[0, pl.ds(d, 1), 8:16] + e1
        alpha_ref[0, pl.ds(g, 1), :] = row0
        alpha_ref[1, pl.ds(g, 1), :] = row1
        return carry

    jax.lax.fori_loop(0, GP, body, 0)


def _pass2_body(x_ref, wv_ref, bv_ref, we_ref, attr_ref, idx_ref,
                alpha_ref, den_ref, out_ref, vs, ss, *, GP, H, C, ED):
    hp = pl.program_id(0)
    c = pl.program_id(1)

    @pl.when(c == 0)
    def _():
        b2 = jnp.concatenate([bv_ref[pl.ds(2 * hp, 1), :],
                              bv_ref[pl.ds(2 * hp + 1, 1), :]], axis=1)
        vs[...] = jnp.dot(x_ref[...], wv_ref[...]) + b2
        ss[...] = jnp.zeros_like(ss)

    @pl.when(jnp.logical_and(hp == 0, c == 0))
    def _():
        out_ref[...] = jnp.zeros_like(out_ref)

    inv_h = 1.0 / H

    def body(g, carry):
        row = idx_ref[pl.ds(g, 1), :]
        arow = attr_ref[pl.ds(g, 1), :]
        ar0 = alpha_ref[0, pl.ds(g, 1), :]
        ar1 = alpha_ref[1, pl.ds(g, 1), :]
        for j in range(8):
            s = row[0, 2 * j]
            d = row[0, 2 * j + 1]
            dn = den_ref[0, pl.ds(d, 1), :]
            a0 = (ar0[0, 16 * j] / (dn[0, 0] + 1e-16)) * inv_h
            a1 = (ar1[0, 16 * j] / (dn[0, 8] + 1e-16)) * inv_h
            vrow = vs[pl.ds(s, 1), :]
            aj = arow[:, 16 * j:16 * j + ED]
            out_ref[pl.ds(d, 1), :] = (out_ref[pl.ds(d, 1), :]
                                       + a0 * vrow[:, 0:C]
                                       + a1 * vrow[:, C:2 * C])
            ss[pl.ds(d, 1), 0:ED] = ss[pl.ds(d, 1), 0:ED] + a0 * aj
            ss[pl.ds(d, 1), ED:2 * ED] = ss[pl.ds(d, 1), ED:2 * ED] + a1 * aj
        return carry

    jax.lax.fori_loop(0, GP, body, 0)

    @pl.when(c == _NCH - 1)
    def _():
        out_ref[...] = (out_ref[...]
                        + jnp.dot(ss[:, 0:ED], we_ref[:, 0:C])
                        + jnp.dot(ss[:, ED:2 * ED], we_ref[:, C:2 * C]))


def _norm_body(aggr_ref, x_ref, ws_ref, bs_ref, batch_ref, gw_ref, gb_ref,
               gms_ref, out_ref, *, G):
    z = aggr_ref[...] + jnp.dot(x_ref[...], ws_ref[...]) + bs_ref[0:1, :]
    n = z.shape[0]
    b = batch_ref[:, 0:1]
    iota = jax.lax.broadcasted_iota(jnp.int32, (n, G), 1)
    oh = (iota == b).astype(jnp.float32)
    cnt = jnp.maximum(jnp.sum(oh, axis=0, keepdims=True), 1.0)
    tot = jax.lax.dot_general(oh, z, (((0,), (0,)), ((), ())))
    mean = tot / cnt.T
    xm = z - jnp.dot(oh, mean) * gms_ref[0:1, :]
    var = jax.lax.dot_general(oh, xm * xm, (((0,), (0,)), ((), ()))) / cnt.T
    std = jnp.sqrt(var + 1e-5)
    y = gw_ref[0:1, :] * xm / jnp.dot(oh, std) + gb_ref[0:1, :]
    out_ref[...] = jnp.where(y >= 0, y, 0.01 * y)


def _layer(x, attr_pk, idx_pk, batch_pad, Wq, bq, Wk, bk, Wv, bv, We, Ws, bs,
           gw, gb, gms, ED):
    N, D = x.shape
    NG = attr_pk.shape[0]
    GP = NG // _NCH
    HC = Wq.shape[1]
    C = Ws.shape[1]
    H = HC // C
    HP = H // 2
    G = 64

    bq2 = bq.reshape(H, C)
    bk2 = bk.reshape(H, C)
    bv2 = bv.reshape(H, C)

    pair_w = pl.BlockSpec((D, 2 * C), lambda h, c: (0, h))
    pair_we = pl.BlockSpec((ED, 2 * C), lambda h, c: (0, h))
    full = lambda a: pl.BlockSpec(a.shape, lambda h, c: (0,) * a.ndim)
    chunk = pl.BlockSpec((GP, 128), lambda h, c: (c, 0))
    chunk_i = pl.BlockSpec((GP, 16), lambda h, c: (c, 0))

    alpha, den = pl.pallas_call(
        functools.partial(_pass1_body, GP=GP, C=C, ED=ED),
        grid=(HP, _NCH),
        in_specs=[full(x), pair_w, full(bq2), pair_w, full(bk2), pair_we,
                  chunk, chunk_i],
        out_specs=[pl.BlockSpec((2, GP, 128), lambda h, c: (h, c, 0)),
                   pl.BlockSpec((1, N, 16), lambda h, c: (h, 0, 0))],
        out_shape=[jax.ShapeDtypeStruct((H, NG, 128), jnp.float32),
                   jax.ShapeDtypeStruct((HP, N, 16), jnp.float32)],
        scratch_shapes=[pltpu.VMEM((N, 2 * C), jnp.float32),
                        pltpu.VMEM((N, 2 * C), jnp.float32),
                        pltpu.VMEM((N, 2 * ED), jnp.float32)],
    )(x, Wq, bq2, Wk, bk2, We, attr_pk, idx_pk)

    aggr = pl.pallas_call(
        functools.partial(_pass2_body, GP=GP, H=H, C=C, ED=ED),
        grid=(HP, _NCH),
        in_specs=[full(x), pair_w, full(bv2), pair_we, chunk, chunk_i,
                  pl.BlockSpec((2, GP, 128), lambda h, c: (h, c, 0)),
                  pl.BlockSpec((1, N, 16), lambda h, c: (h, 0, 0))],
        out_specs=pl.BlockSpec((N, C), lambda h, c: (0, 0)),
        out_shape=jax.ShapeDtypeStruct((N, C), jnp.float32),
        scratch_shapes=[pltpu.VMEM((N, 2 * C), jnp.float32),
                        pltpu.VMEM((N, 2 * ED), jnp.float32)],
    )(x, Wv, bv2, We, attr_pk, idx_pk, alpha, den)

    vec = lambda v: jnp.broadcast_to(v.reshape(1, -1), (8, v.shape[0]))
    out = pl.pallas_call(
        functools.partial(_norm_body, G=G),
        out_shape=jax.ShapeDtypeStruct((N, C), jnp.float32),
    )(aggr, x, Ws, vec(bs), batch_pad, vec(gw), vec(gb), vec(gms))
    return out


def kernel(x, index, attr, batch, params):
    E = index.shape[1]
    N = x.shape[0]
    ED = attr.shape[1]
    attr_pk = attr.reshape(E // 8, 8 * ED)
    idx_pk = jnp.stack([index[0], index[1]], axis=1).reshape(E // 8, 16)
    batch_pad = jnp.broadcast_to(batch.reshape(N, 1), (N, 8)).astype(jnp.int32)

    h = x
    for l in ('1', '2'):
        h = _layer(h, attr_pk, idx_pk, batch_pad,
                   params['Wq' + l], params['bq' + l],
                   params['Wk' + l], params['bk' + l],
                   params['Wv' + l], params['bv' + l],
                   params['We' + l], params['Ws' + l], params['bs' + l],
                   params['gw' + l], params['gb' + l], params['gms' + l], ED)
    return h
